# baseline (speedup 1.0000x reference)
import functools
import math

import jax
import jax.numpy as jnp
from jax.experimental import pallas as pl
from jax.experimental.pallas import tpu as pltpu

_EPS = 1e-6
_MAX_POSITIONS = 10000.0


def _fused_kernel(x_ref, noise_ref,
                  m0_ref, mb0_ref, m1_ref, mb1_ref,
                  lin0_ref, lb0_ref, gw0_ref, gb0_ref,
                  qkv0_ref, qb0_ref, prj0_ref, pb0_ref,
                  lin1_ref, lb1_ref, gw1_ref, gb1_ref,
                  qkv1_ref, qb1_ref, prj1_ref, pb1_ref,
                  wo_ref, bo_ref, o_ref,
                  s_map0, s_map1, s_lin0, s_qkv0, s_prj0,
                  s_lin1, s_qkv1, s_prj1, s_out, s_gn, s_att,
                  s_dn, s_up, s_e0,
                  l0, l1, l2, l3, l4, l5, l6, l7, l8, sem,
                  *, bt, t, gr, inv_gd, scale, nls):
    f32 = jnp.float32
    bf16 = jnp.bfloat16
    C = prj0_ref.shape[0]
    _NEG_LOG_STEP = nls

    hbm_refs = [m0_ref, m1_ref, lin0_ref, qkv0_ref, prj0_ref,
                lin1_ref, qkv1_ref, prj1_ref, wo_ref]
    lands = [l0, l1, l2, l3, l4, l5, l6, l7, l8]

    def _copy(j):
        return pltpu.make_async_copy(hbm_refs[j], lands[j], sem.at[j])

    first = pl.program_id(0) == 0

    def wprep(j, dst):
        @pl.when(first)
        def _():
            _copy(j).wait()
            dst[...] = lands[j][...].T.astype(bf16)

    @pl.when(first)
    def _prep():
        for j in range(len(hbm_refs)):
            _copy(j).start()
        nsteps, nbt, nc = s_e0.shape
        f_half = nc // 2
        jf = jax.lax.broadcasted_iota(jnp.int32, (1, f_half), 1).astype(f32)
        freqs = jnp.exp(jf * _NEG_LOG_STEP)
        phase = noise_ref[...].T * freqs
        e0 = jnp.concatenate([jnp.sin(phase), jnp.cos(phase)], axis=1)
        s_e0[...] = e0.reshape(nsteps, nbt, nc)
        gcols = jnp.concatenate([gw0_ref[...].T, gb0_ref[...].T,
                                 gw1_ref[...].T, gb1_ref[...].T], axis=1)
        s_gn[...] = jnp.concatenate([gcols] * bt, axis=0)
        m = bt * t
        ngrp = m // gr
        gi = jax.lax.broadcasted_iota(jnp.int32, (ngrp, m), 0)
        gj = jax.lax.broadcasted_iota(jnp.int32, (ngrp, m), 1) // gr
        s_dn[...] = (gi == gj).astype(jnp.float32)
        ui = jax.lax.broadcasted_iota(jnp.int32, (m, ngrp), 0) // gr
        uj = jax.lax.broadcasted_iota(jnp.int32, (m, ngrp), 1)
        s_up[...] = (ui == uj).astype(jnp.float32)

    def mm(a, w):
        return jax.lax.dot_general(a.astype(bf16), w,
                                   (((1,), (0,)), ((), ())),
                                   preferred_element_type=f32)

    wprep(0, s_map0)
    e = mm(s_e0[pl.program_id(0)], s_map0[...]) + mb0_ref[...]
    e = e * jax.nn.sigmoid(e)
    wprep(1, s_map1)
    e = mm(e, s_map1[...]) + mb1_ref[...]
    emb = e * jax.nn.sigmoid(e)

    Adn = s_dn[...]
    Aup = s_up[...]

    def linear_block(xf, s_lin, lb_ref, s_qkv, qb_ref, s_prj, pb_ref,
                     gw, gb, cin, jl, jq, jp):
        wprep(jl, s_lin)
        wfull = s_lin[...]
        embc = mm(emb, wfull[cin:]) + lb_ref[...]
        h = mm(xf, wfull[:cin])
        h = h.reshape(bt, t, C) + embc[:, None, :]
        h = jnp.maximum(h, 0.0).reshape(bt * t, C)

        s1 = jnp.sum(h, axis=1, keepdims=True)
        s2 = jnp.sum(h * h, axis=1, keepdims=True)
        gs = jax.lax.dot_general(Adn, jnp.concatenate([s1, s2], axis=1),
                                 (((1,), (0,)), ((), ())),
                                 preferred_element_type=f32)
        g = jax.lax.dot_general(Aup, gs, (((1,), (0,)), ((), ())),
                                preferred_element_type=f32) * inv_gd
        mu = g[:, 0:1]
        var = g[:, 1:2] - mu * mu
        hn = (h - mu) * jax.lax.rsqrt(var + _EPS) * gw + gb

        wprep(jq, s_qkv)
        z = mm(hn, s_qkv[...]) + qb_ref[...]
        for b in range(bt):
            zb = z[b * t:(b + 1) * t]
            qb = zb[:, :C].astype(bf16)
            kb = zb[:, C:2 * C].astype(bf16)
            vb = zb[:, 2 * C:].astype(bf16)
            logits = jax.lax.dot_general(qb, kb, (((1,), (1,)), ((), ())),
                                         preferred_element_type=f32)
            logits = logits * scale
            m = jnp.max(logits, axis=-1, keepdims=True)
            p = jnp.exp(logits - m)
            w = (p / jnp.sum(p, axis=-1, keepdims=True)).astype(bf16)
            s_att[b * t:(b + 1) * t, :] = jax.lax.dot_general(
                w, vb, (((1,), (0,)), ((), ())),
                preferred_element_type=f32).astype(bf16)
        a = s_att[...]
        wprep(jp, s_prj)
        return (jax.lax.dot_general(a, s_prj[...], (((1,), (0,)), ((), ())),
                                    preferred_element_type=f32)
                + pb_ref[...] + hn).astype(bf16)

    xin = x_ref[...]
    d_in = xin.shape[2]
    x1 = linear_block(xin.reshape(bt * t, d_in), s_lin0, lb0_ref,
                      s_qkv0, qb0_ref, s_prj0, pb0_ref,
                      s_gn[:, 0:1], s_gn[:, 1:2], d_in, 2, 3, 4)
    x2 = linear_block(x1, s_lin1, lb1_ref,
                      s_qkv1, qb1_ref, s_prj1, pb1_ref,
                      s_gn[:, 2:3], s_gn[:, 3:4], C, 5, 6, 7)
    wprep(8, s_out)
    out = mm(x2, s_out[...]) + bo_ref[...]
    o_ref[...] = out.reshape(bt, t, out.shape[1])


def kernel(x, noise_labels, map0_w, map0_b, map1_w, map1_b, out_w, out_b,
           block0_lin_w, block0_lin_b, block0_gn_w, block0_gn_b,
           block0_qkv_w, block0_qkv_b, block0_proj_w, block0_proj_b,
           block1_lin_w, block1_lin_b, block1_gn_w, block1_gn_b,
           block1_qkv_w, block1_qkv_b, block1_proj_w, block1_proj_b):
    B, T, D = x.shape
    E = map0_w.shape[0]
    NC = map0_w.shape[1]
    C = block0_lin_w.shape[0]
    Dout = out_w.shape[0]
    num_groups = min(8, T // 4)
    group_rows = T // num_groups
    inv_gd = 1.0 / (group_rows * C)
    scale = 1.0 / math.sqrt(C)

    F = NC // 2
    nls = -math.log(_MAX_POSITIONS) / (F - 1)

    BT = next(bt for bt in (8, 4, 3, 2, 1) if B % bt == 0)
    steps = B // BT
    M = BT * T

    args = [x, noise_labels.reshape(1, B),
            map0_w, map0_b.reshape(1, E), map1_w, map1_b.reshape(1, E),
            block0_lin_w, block0_lin_b.reshape(1, C),
            block0_gn_w.reshape(1, T), block0_gn_b.reshape(1, T),
            block0_qkv_w, block0_qkv_b.reshape(1, 3 * C),
            block0_proj_w, block0_proj_b.reshape(1, C),
            block1_lin_w, block1_lin_b.reshape(1, C),
            block1_gn_w.reshape(1, T), block1_gn_b.reshape(1, T),
            block1_qkv_w, block1_qkv_b.reshape(1, 3 * C),
            block1_proj_w, block1_proj_b.reshape(1, C),
            out_w, out_b.reshape(1, Dout)]

    def const_spec(a):
        nd = a.ndim
        return pl.BlockSpec(a.shape, lambda i, _nd=nd: (0,) * _nd)

    any_spec = pl.BlockSpec(memory_space=pl.ANY)
    weight_idx = {2, 4, 6, 10, 12, 14, 18, 20, 22}
    in_specs = ([pl.BlockSpec((BT, T, D), lambda i: (i, 0, 0))]
                + [any_spec if j in weight_idx else const_spec(a)
                   for j, a in enumerate(args) if j >= 1])

    bf16 = jnp.bfloat16
    scratch_shapes = [
        pltpu.VMEM((NC, E), bf16),
        pltpu.VMEM((E, E), bf16),
        pltpu.VMEM((D + E, C), bf16),
        pltpu.VMEM((C, 3 * C), bf16),
        pltpu.VMEM((C, C), bf16),
        pltpu.VMEM((C + E, C), bf16),
        pltpu.VMEM((C, 3 * C), bf16),
        pltpu.VMEM((C, C), bf16),
        pltpu.VMEM((C, Dout), bf16),
        pltpu.VMEM((M, 4), jnp.float32),
        pltpu.VMEM((M, C), bf16),
        pltpu.VMEM((M // group_rows, M), jnp.float32),
        pltpu.VMEM((M, M // group_rows), jnp.float32),
        pltpu.VMEM((steps, BT, NC), jnp.float32),
        pltpu.VMEM((E, NC), jnp.float32),
        pltpu.VMEM((E, E), jnp.float32),
        pltpu.VMEM((C, D + E), jnp.float32),
        pltpu.VMEM((3 * C, C), jnp.float32),
        pltpu.VMEM((C, C), jnp.float32),
        pltpu.VMEM((C, C + E), jnp.float32),
        pltpu.VMEM((3 * C, C), jnp.float32),
        pltpu.VMEM((C, C), jnp.float32),
        pltpu.VMEM((Dout, C), jnp.float32),
        pltpu.SemaphoreType.DMA((9,)),
    ]

    fn = functools.partial(_fused_kernel, bt=BT, t=T, gr=group_rows,
                           inv_gd=inv_gd, scale=scale, nls=nls)
    out = pl.pallas_call(
        fn,
        grid=(steps,),
        in_specs=in_specs,
        out_specs=pl.BlockSpec((BT, T, Dout), lambda i: (i, 0, 0)),
        out_shape=jax.ShapeDtypeStruct((B, T, Dout), jnp.float32),
        scratch_shapes=scratch_shapes,
        compiler_params=pltpu.CompilerParams(
            dimension_semantics=("arbitrary",)),
    )(*args)
    return out


# speedup vs baseline: 1.0260x; 1.0260x over previous
import functools
import math

import jax
import jax.numpy as jnp
from jax.experimental import pallas as pl
from jax.experimental.pallas import tpu as pltpu

_EPS = 1e-6
_MAX_POSITIONS = 10000.0


def _fused_kernel(x_ref, noise_ref,
                  m0_ref, mb0_ref, m1_ref, mb1_ref,
                  lin0_ref, lb0_ref, gw0_ref, gb0_ref,
                  qkv0_ref, qb0_ref, prj0_ref, pb0_ref,
                  lin1_ref, lb1_ref, gw1_ref, gb1_ref,
                  qkv1_ref, qb1_ref, prj1_ref, pb1_ref,
                  wo_ref, bo_ref, o_ref,
                  s_map0, s_map1, s_lin0, s_qkv0, s_prj0,
                  s_lin1, s_qkv1, s_prj1, s_out, s_gn, s_att,
                  s_dn, s_up, s_e0,
                  l0, l1, l2, l3, l4, l5, l6, l7, l8, sem,
                  *, bt, t, gr, inv_gd, scale, nls):
    f32 = jnp.float32
    bf16 = jnp.bfloat16
    C = prj0_ref.shape[0]
    _NEG_LOG_STEP = nls

    hbm_refs = [m0_ref, m1_ref, lin0_ref, qkv0_ref, prj0_ref,
                lin1_ref, qkv1_ref, prj1_ref, wo_ref]
    lands = [l0, l1, l2, l3, l4, l5, l6, l7, l8]

    def _copy(j):
        return pltpu.make_async_copy(hbm_refs[j], lands[j], sem.at[j])

    @pl.when(pl.program_id(0) == 0)
    def _prep():
        for j in range(len(hbm_refs)):
            _copy(j).start()

        def landed(j):
            _copy(j).wait()
            return lands[j][...]

        s_map0[...] = landed(0).T.astype(bf16)
        s_map1[...] = landed(1).T.astype(bf16)
        s_lin0[...] = landed(2).T.astype(bf16)
        s_qkv0[...] = landed(3).T.astype(bf16)
        s_prj0[...] = landed(4).T.astype(bf16)
        s_lin1[...] = landed(5).T.astype(bf16)
        s_qkv1[...] = landed(6).T.astype(bf16)
        s_prj1[...] = landed(7).T.astype(bf16)
        s_out[...] = landed(8).T.astype(bf16)
        nsteps, nbt, nc = s_e0.shape
        f_half = nc // 2
        jf = jax.lax.broadcasted_iota(jnp.int32, (1, f_half), 1).astype(f32)
        freqs = jnp.exp(jf * _NEG_LOG_STEP)
        phase = noise_ref[...].T * freqs
        e0 = jnp.concatenate([jnp.sin(phase), jnp.cos(phase)], axis=1)
        s_e0[...] = e0.reshape(nsteps, nbt, nc)
        gcols = jnp.concatenate([gw0_ref[...].T, gb0_ref[...].T,
                                 gw1_ref[...].T, gb1_ref[...].T], axis=1)
        s_gn[...] = jnp.concatenate([gcols] * bt, axis=0)
        m = bt * t
        ngrp = m // gr
        gi = jax.lax.broadcasted_iota(jnp.int32, (ngrp, m), 0)
        gj = jax.lax.broadcasted_iota(jnp.int32, (ngrp, m), 1) // gr
        s_dn[...] = (gi == gj).astype(jnp.float32)
        ui = jax.lax.broadcasted_iota(jnp.int32, (m, ngrp), 0) // gr
        uj = jax.lax.broadcasted_iota(jnp.int32, (m, ngrp), 1)
        s_up[...] = (ui == uj).astype(jnp.float32)

    def mm(a, w):
        return jax.lax.dot_general(a.astype(bf16), w,
                                   (((1,), (0,)), ((), ())),
                                   preferred_element_type=f32)

    e = mm(s_e0[pl.program_id(0)], s_map0[...]) + mb0_ref[...]
    e = e * jax.nn.sigmoid(e)
    e = mm(e, s_map1[...]) + mb1_ref[...]
    emb = e * jax.nn.sigmoid(e)

    Adn = s_dn[...]
    Aup = s_up[...]

    def linear_block(xf, s_lin, lb_ref, s_qkv, qb_ref, s_prj, pb_ref,
                     gw, gb, cin):
        wfull = s_lin[...]
        embc = mm(emb, wfull[cin:]) + lb_ref[...]
        h = mm(xf, wfull[:cin])
        h = h.reshape(bt, t, C) + embc[:, None, :]
        h = jnp.maximum(h, 0.0).reshape(bt * t, C)

        s1 = jnp.sum(h, axis=1, keepdims=True)
        s2 = jnp.sum(h * h, axis=1, keepdims=True)
        gs = jax.lax.dot_general(Adn, jnp.concatenate([s1, s2], axis=1),
                                 (((1,), (0,)), ((), ())),
                                 preferred_element_type=f32)
        g = jax.lax.dot_general(Aup, gs, (((1,), (0,)), ((), ())),
                                preferred_element_type=f32) * inv_gd
        mu = g[:, 0:1]
        var = g[:, 1:2] - mu * mu
        hn = (h - mu) * jax.lax.rsqrt(var + _EPS) * gw + gb

        z = mm(hn, s_qkv[...]) + qb_ref[...]
        for b in range(bt):
            zb = z[b * t:(b + 1) * t]
            qb = zb[:, :C].astype(bf16)
            kb = zb[:, C:2 * C].astype(bf16)
            vb = zb[:, 2 * C:].astype(bf16)
            logits = jax.lax.dot_general(qb, kb, (((1,), (1,)), ((), ())),
                                         preferred_element_type=f32)
            logits = logits * scale
            m = jnp.max(logits, axis=-1, keepdims=True)
            p = jnp.exp(logits - m)
            w = (p / jnp.sum(p, axis=-1, keepdims=True)).astype(bf16)
            s_att[b * t:(b + 1) * t, :] = jax.lax.dot_general(
                w, vb, (((1,), (0,)), ((), ())),
                preferred_element_type=f32).astype(bf16)
        a = s_att[...]
        return (jax.lax.dot_general(a, s_prj[...], (((1,), (0,)), ((), ())),
                                    preferred_element_type=f32)
                + pb_ref[...] + hn).astype(bf16)

    xin = x_ref[...]
    d_in = xin.shape[2]
    x1 = linear_block(xin.reshape(bt * t, d_in), s_lin0, lb0_ref,
                      s_qkv0, qb0_ref, s_prj0, pb0_ref,
                      s_gn[:, 0:1], s_gn[:, 1:2], d_in)
    x2 = linear_block(x1, s_lin1, lb1_ref,
                      s_qkv1, qb1_ref, s_prj1, pb1_ref,
                      s_gn[:, 2:3], s_gn[:, 3:4], C)
    out = mm(x2, s_out[...]) + bo_ref[...]
    o_ref[...] = out.reshape(bt, t, out.shape[1])


def kernel(x, noise_labels, map0_w, map0_b, map1_w, map1_b, out_w, out_b,
           block0_lin_w, block0_lin_b, block0_gn_w, block0_gn_b,
           block0_qkv_w, block0_qkv_b, block0_proj_w, block0_proj_b,
           block1_lin_w, block1_lin_b, block1_gn_w, block1_gn_b,
           block1_qkv_w, block1_qkv_b, block1_proj_w, block1_proj_b):
    B, T, D = x.shape
    E = map0_w.shape[0]
    NC = map0_w.shape[1]
    C = block0_lin_w.shape[0]
    Dout = out_w.shape[0]
    num_groups = min(8, T // 4)
    group_rows = T // num_groups
    inv_gd = 1.0 / (group_rows * C)
    scale = 1.0 / math.sqrt(C)

    F = NC // 2
    nls = -math.log(_MAX_POSITIONS) / (F - 1)

    BT = next(bt for bt in (8, 4, 3, 2, 1) if B % bt == 0)
    steps = B // BT
    M = BT * T

    args = [x, noise_labels.reshape(1, B),
            map0_w, map0_b.reshape(1, E), map1_w, map1_b.reshape(1, E),
            block0_lin_w, block0_lin_b.reshape(1, C),
            block0_gn_w.reshape(1, T), block0_gn_b.reshape(1, T),
            block0_qkv_w, block0_qkv_b.reshape(1, 3 * C),
            block0_proj_w, block0_proj_b.reshape(1, C),
            block1_lin_w, block1_lin_b.reshape(1, C),
            block1_gn_w.reshape(1, T), block1_gn_b.reshape(1, T),
            block1_qkv_w, block1_qkv_b.reshape(1, 3 * C),
            block1_proj_w, block1_proj_b.reshape(1, C),
            out_w, out_b.reshape(1, Dout)]

    def const_spec(a):
        nd = a.ndim
        return pl.BlockSpec(a.shape, lambda i, _nd=nd: (0,) * _nd)

    any_spec = pl.BlockSpec(memory_space=pl.ANY)
    weight_idx = {2, 4, 6, 10, 12, 14, 18, 20, 22}
    in_specs = ([pl.BlockSpec((BT, T, D), lambda i: (i, 0, 0))]
                + [any_spec if j in weight_idx else const_spec(a)
                   for j, a in enumerate(args) if j >= 1])

    bf16 = jnp.bfloat16
    scratch_shapes = [
        pltpu.VMEM((NC, E), bf16),
        pltpu.VMEM((E, E), bf16),
        pltpu.VMEM((D + E, C), bf16),
        pltpu.VMEM((C, 3 * C), bf16),
        pltpu.VMEM((C, C), bf16),
        pltpu.VMEM((C + E, C), bf16),
        pltpu.VMEM((C, 3 * C), bf16),
        pltpu.VMEM((C, C), bf16),
        pltpu.VMEM((C, Dout), bf16),
        pltpu.VMEM((M, 4), jnp.float32),
        pltpu.VMEM((M, C), bf16),
        pltpu.VMEM((M // group_rows, M), jnp.float32),
        pltpu.VMEM((M, M // group_rows), jnp.float32),
        pltpu.VMEM((steps, BT, NC), jnp.float32),
        pltpu.VMEM((E, NC), jnp.float32),
        pltpu.VMEM((E, E), jnp.float32),
        pltpu.VMEM((C, D + E), jnp.float32),
        pltpu.VMEM((3 * C, C), jnp.float32),
        pltpu.VMEM((C, C), jnp.float32),
        pltpu.VMEM((C, C + E), jnp.float32),
        pltpu.VMEM((3 * C, C), jnp.float32),
        pltpu.VMEM((C, C), jnp.float32),
        pltpu.VMEM((Dout, C), jnp.float32),
        pltpu.SemaphoreType.DMA((9,)),
    ]

    fn = functools.partial(_fused_kernel, bt=BT, t=T, gr=group_rows,
                           inv_gd=inv_gd, scale=scale, nls=nls)
    out = pl.pallas_call(
        fn,
        grid=(steps,),
        in_specs=in_specs,
        out_specs=pl.BlockSpec((BT, T, Dout), lambda i: (i, 0, 0)),
        out_shape=jax.ShapeDtypeStruct((B, T, Dout), jnp.float32),
        scratch_shapes=scratch_shapes,
        compiler_params=pltpu.CompilerParams(
            dimension_semantics=("arbitrary",)),
    )(*args)
    return out
